# Initial kernel scaffold; baseline (speedup 1.0000x reference)
import jax
import jax.numpy as jnp
from jax.experimental import pallas as pl


def kernel(x, idx, queue_x, age):
    raise NotImplementedError("write your pallas kernel here")


# speedup vs baseline: 4.0040x; 4.0040x over previous
import functools

import jax
import jax.numpy as jnp
from jax import lax
from jax.experimental import pallas as pl
from jax.experimental.pallas import tpu as pltpu
from jax.experimental.pallas import tpu_sc as plsc

_SIZE = 65536
_DIM = 256
_ROWS = 2048
_QB = 512
_EPS = 1e-12


def _topk_body(x_ref, q_ref, idx_out, sim_out, bestv, nx):
    pid = pl.program_id(0)

    @pl.when(pid == 0)
    def _():
        xv = x_ref[...]
        xnorm = jnp.maximum(jnp.sqrt(jnp.sum(xv * xv, axis=1, keepdims=True)), _EPS)
        nx[...] = xv / xnorm

    q = q_ref[...]
    qnorm = jnp.maximum(jnp.sqrt(jnp.sum(q * q, axis=1, keepdims=True)), _EPS)
    qn = q / qnorm
    t = lax.dot_general(nx[...], qn, (((1,), (1,)), ((), ())),
                        preferred_element_type=jnp.float32)
    m = jnp.max(t, axis=1, keepdims=True)
    col = lax.broadcasted_iota(jnp.int32, t.shape, 1) + pid * _QB
    arg = jnp.min(jnp.where(t == m, col, _SIZE), axis=1, keepdims=True)

    @pl.when(pid == 0)
    def _():
        bestv[...] = m
        idx_out[...] = arg

    @pl.when(pid != 0)
    def _():
        bv = bestv[...]
        better = m > bv
        bestv[...] = jnp.where(better, m, bv)
        idx_out[...] = jnp.where(better, arg, idx_out[...])

    @pl.when(pid == pl.num_programs(0) - 1)
    def _():
        sim_out[0, 0] = jnp.sum(bestv[...]) / _ROWS


_topk = pl.pallas_call(
    _topk_body,
    grid=(_SIZE // _QB,),
    in_specs=[
        pl.BlockSpec((_ROWS, _DIM), lambda i: (0, 0)),
        pl.BlockSpec((_QB, _DIM), lambda i: (i, 0)),
    ],
    out_specs=[
        pl.BlockSpec((_ROWS, 1), lambda i: (0, 0)),
        pl.BlockSpec(memory_space=pltpu.SMEM),
    ],
    out_shape=[
        jax.ShapeDtypeStruct((_ROWS, 1), jnp.int32),
        jax.ShapeDtypeStruct((1, 1), jnp.float32),
    ],
    scratch_shapes=[pltpu.VMEM((_ROWS, 1), jnp.float32),
                    pltpu.VMEM((_ROWS, _DIM), jnp.float32)],
)


_NC, _NS, _L = 2, 16, 16
_NW = _NC * _NS
_BPW = _ROWS // _NW


@functools.partial(
    pl.kernel,
    mesh=plsc.VectorSubcoreMesh(core_axis_name="c", subcore_axis_name="s"),
    out_type=[
        jax.ShapeDtypeStruct((_ROWS, _DIM), jnp.float32),
        jax.ShapeDtypeStruct((_ROWS,), jnp.int32),
    ],
    scratch_types=[
        pltpu.VMEM((_BPW,), jnp.int32),
        pltpu.VMEM((_BPW, _DIM), jnp.float32),
        pltpu.VMEM((_BPW,), jnp.int32),
        pltpu.SemaphoreType.DMA,
        pltpu.SemaphoreType.DMA,
    ],
)
def _gather(table_hbm, idx_hbm, age_hbm, rows_out, age_out,
            idx_v, rows_v, ageo_v, sem, sem2):
    wid = lax.axis_index("s") * _NC + lax.axis_index("c")
    base = wid * _BPW
    pltpu.sync_copy(idx_hbm.at[pl.ds(base, _BPW)], idx_v)
    cp1 = pltpu.async_copy(table_hbm.at[idx_v], rows_v, sem)
    cp2 = pltpu.async_copy(age_hbm.at[idx_v], ageo_v, sem2)
    cp1.wait()
    cp2.wait()
    pltpu.sync_copy(rows_v, rows_out.at[pl.ds(base, _BPW)])
    pltpu.sync_copy(ageo_v, age_out.at[pl.ds(base, _BPW)])


def kernel(x, idx, queue_x, age):
    del idx
    best2, simmean = _topk(x, queue_x)
    best_idx = best2.reshape(_ROWS)
    nn_x, age_g = _gather(queue_x, best_idx, age)
    nn_similarity = simmean[0, 0]
    nn_age = jnp.mean(age_g.astype(jnp.float32))
    return nn_x, nn_similarity, nn_age


# speedup vs baseline: 5.9281x; 1.4805x over previous
import functools

import jax
import jax.numpy as jnp
from jax import lax
from jax.experimental import pallas as pl
from jax.experimental.pallas import tpu as pltpu
from jax.experimental.pallas import tpu_sc as plsc

_SIZE = 65536
_DIM = 256
_ROWS = 2048
_QB = 1024
_EPS = 1e-12


def _topk_body(x_ref, q_ref, idx_out, sim_out, bestv, besti, nx):
    pid = pl.program_id(0)

    @pl.when(pid == 0)
    def _():
        xv = x_ref[...]
        xnorm = jnp.maximum(jnp.sqrt(jnp.sum(xv * xv, axis=1, keepdims=True)), _EPS)
        nx[...] = xv / xnorm

    q = q_ref[...]
    qnorm = jnp.maximum(jnp.sqrt(jnp.sum(q * q, axis=1, keepdims=True)), _EPS)
    qn = q / qnorm
    t = lax.dot_general(nx[...], qn, (((1,), (1,)), ((), ())),
                        preferred_element_type=jnp.float32)
    m = jnp.max(t, axis=1, keepdims=True)
    col = (lax.broadcasted_iota(jnp.int32, t.shape, 1).astype(jnp.float32)
           + (pid * _QB).astype(jnp.float32))
    arg = jnp.min(jnp.where(t == m, col, jnp.float32(_SIZE)), axis=1,
                  keepdims=True)

    @pl.when(pid == 0)
    def _():
        bestv[...] = m
        besti[...] = arg

    @pl.when(pid != 0)
    def _():
        bv = bestv[...]
        better = m > bv
        bestv[...] = jnp.where(better, m, bv)
        besti[...] = jnp.where(better, arg, besti[...])

    @pl.when(pid == pl.num_programs(0) - 1)
    def _():
        idx_out[...] = besti[...].astype(jnp.int32)
        sim_out[0, 0] = jnp.sum(bestv[...]) / _ROWS


_topk = pl.pallas_call(
    _topk_body,
    grid=(_SIZE // _QB,),
    in_specs=[
        pl.BlockSpec((_ROWS, _DIM), lambda i: (0, 0)),
        pl.BlockSpec((_QB, _DIM), lambda i: (i, 0)),
    ],
    out_specs=[
        pl.BlockSpec((_ROWS, 1), lambda i: (0, 0)),
        pl.BlockSpec(memory_space=pltpu.SMEM),
    ],
    out_shape=[
        jax.ShapeDtypeStruct((_ROWS, 1), jnp.int32),
        jax.ShapeDtypeStruct((1, 1), jnp.float32),
    ],
    scratch_shapes=[pltpu.VMEM((_ROWS, 1), jnp.float32),
                    pltpu.VMEM((_ROWS, 1), jnp.float32),
                    pltpu.VMEM((_ROWS, _DIM), jnp.float32)],
)


_NC, _NS, _L = 2, 16, 16
_NW = _NC * _NS
_BPW = _ROWS // _NW


@functools.partial(
    pl.kernel,
    mesh=plsc.VectorSubcoreMesh(core_axis_name="c", subcore_axis_name="s"),
    out_type=[
        jax.ShapeDtypeStruct((_ROWS, _DIM), jnp.float32),
        jax.ShapeDtypeStruct((_ROWS,), jnp.int32),
    ],
    scratch_types=[
        pltpu.VMEM((_BPW,), jnp.int32),
        pltpu.VMEM((_BPW, _DIM), jnp.float32),
        pltpu.VMEM((_BPW,), jnp.int32),
        pltpu.SemaphoreType.DMA,
        pltpu.SemaphoreType.DMA,
    ],
)
def _gather(table_hbm, idx_hbm, age_hbm, rows_out, age_out,
            idx_v, rows_v, ageo_v, sem, sem2):
    wid = lax.axis_index("s") * _NC + lax.axis_index("c")
    base = wid * _BPW
    pltpu.sync_copy(idx_hbm.at[pl.ds(base, _BPW)], idx_v)
    cp1 = pltpu.async_copy(table_hbm.at[idx_v], rows_v, sem)
    cp2 = pltpu.async_copy(age_hbm.at[idx_v], ageo_v, sem2)
    cp1.wait()
    cp2.wait()
    pltpu.sync_copy(rows_v, rows_out.at[pl.ds(base, _BPW)])
    pltpu.sync_copy(ageo_v, age_out.at[pl.ds(base, _BPW)])


def kernel(x, idx, queue_x, age):
    del idx
    best2, simmean = _topk(x, queue_x)
    best_idx = best2.reshape(_ROWS)
    nn_x, age_g = _gather(queue_x, best_idx, age)
    nn_similarity = simmean[0, 0]
    nn_age = jnp.mean(age_g.astype(jnp.float32))
    return nn_x, nn_similarity, nn_age


# speedup vs baseline: 6.8629x; 1.1577x over previous
import functools

import jax
import jax.numpy as jnp
from jax import lax
from jax.experimental import pallas as pl
from jax.experimental.pallas import tpu as pltpu
from jax.experimental.pallas import tpu_sc as plsc

_SIZE = 65536
_DIM = 256
_ROWS = 2048
_QB = 4096
_EPS = 1e-12


def _topk_body(x_ref, q_ref, idx_out, sim_out, bestv, besti, nx):
    pid = pl.program_id(0)

    @pl.when(pid == 0)
    def _():
        xv = x_ref[...]
        xnorm = jnp.maximum(jnp.sqrt(jnp.sum(xv * xv, axis=1, keepdims=True)), _EPS)
        nx[...] = xv / xnorm

    q = q_ref[...]
    qnorm = jnp.maximum(jnp.sqrt(jnp.sum(q * q, axis=1, keepdims=True)), _EPS)
    qn = q / qnorm
    t = lax.dot_general(nx[...], qn, (((1,), (1,)), ((), ())),
                        preferred_element_type=jnp.float32)
    m = jnp.max(t, axis=1, keepdims=True)
    col = (lax.broadcasted_iota(jnp.int32, t.shape, 1).astype(jnp.float32)
           + (pid * _QB).astype(jnp.float32))
    arg = jnp.min(jnp.where(t == m, col, jnp.float32(_SIZE)), axis=1,
                  keepdims=True)

    @pl.when(pid == 0)
    def _():
        bestv[...] = m
        besti[...] = arg

    @pl.when(pid != 0)
    def _():
        bv = bestv[...]
        better = m > bv
        bestv[...] = jnp.where(better, m, bv)
        besti[...] = jnp.where(better, arg, besti[...])

    @pl.when(pid == pl.num_programs(0) - 1)
    def _():
        idx_out[...] = besti[...].astype(jnp.int32)
        sim_out[0, 0] = jnp.sum(bestv[...]) / _ROWS


_topk = pl.pallas_call(
    _topk_body,
    grid=(_SIZE // _QB,),
    in_specs=[
        pl.BlockSpec((_ROWS, _DIM), lambda i: (0, 0)),
        pl.BlockSpec((_QB, _DIM), lambda i: (i, 0)),
    ],
    out_specs=[
        pl.BlockSpec((_ROWS, 1), lambda i: (0, 0)),
        pl.BlockSpec(memory_space=pltpu.SMEM),
    ],
    out_shape=[
        jax.ShapeDtypeStruct((_ROWS, 1), jnp.int32),
        jax.ShapeDtypeStruct((1, 1), jnp.float32),
    ],
    scratch_shapes=[pltpu.VMEM((_ROWS, 1), jnp.float32),
                    pltpu.VMEM((_ROWS, 1), jnp.float32),
                    pltpu.VMEM((_ROWS, _DIM), jnp.float32)],
)


_NC, _NS, _L = 2, 16, 16
_NW = _NC * _NS
_BPW = _ROWS // _NW


@functools.partial(
    pl.kernel,
    mesh=plsc.VectorSubcoreMesh(core_axis_name="c", subcore_axis_name="s"),
    out_type=[
        jax.ShapeDtypeStruct((_ROWS, _DIM), jnp.float32),
        jax.ShapeDtypeStruct((_ROWS,), jnp.int32),
    ],
    scratch_types=[
        pltpu.VMEM((_BPW,), jnp.int32),
        pltpu.VMEM((_BPW, _DIM), jnp.float32),
        pltpu.VMEM((_BPW,), jnp.int32),
        pltpu.SemaphoreType.DMA,
        pltpu.SemaphoreType.DMA,
    ],
)
def _gather(table_hbm, idx_hbm, age_hbm, rows_out, age_out,
            idx_v, rows_v, ageo_v, sem, sem2):
    wid = lax.axis_index("s") * _NC + lax.axis_index("c")
    base = wid * _BPW
    pltpu.sync_copy(idx_hbm.at[pl.ds(base, _BPW)], idx_v)
    cp1 = pltpu.async_copy(table_hbm.at[idx_v], rows_v, sem)
    cp2 = pltpu.async_copy(age_hbm.at[idx_v], ageo_v, sem2)
    cp1.wait()
    cp2.wait()
    pltpu.sync_copy(rows_v, rows_out.at[pl.ds(base, _BPW)])
    pltpu.sync_copy(ageo_v, age_out.at[pl.ds(base, _BPW)])


def kernel(x, idx, queue_x, age):
    del idx
    best2, simmean = _topk(x, queue_x)
    best_idx = best2.reshape(_ROWS)
    nn_x, age_g = _gather(queue_x, best_idx, age)
    nn_similarity = simmean[0, 0]
    nn_age = jnp.mean(age_g.astype(jnp.float32))
    return nn_x, nn_similarity, nn_age
